# baseline (speedup 1.0000x reference)
import functools

import jax
import jax.numpy as jnp
from jax import lax
from jax.experimental import pallas as pl
from jax.experimental.pallas import tpu as pltpu
from jax.experimental.pallas import tpu_sc as plsc

B = 16384
D = 32
NC = 2
NS = 16
NW = NC * NS
B_PER_W = B // NW
IDX_CHUNK = 128
N_CHUNKS = B_PER_W // IDX_CHUNK


def _make_sc_gather():
    mesh = plsc.VectorSubcoreMesh(core_axis_name="c", subcore_axis_name="s")

    @functools.partial(
        pl.kernel,
        out_type=(
            jax.ShapeDtypeStruct((B, D), jnp.float32),
            jax.ShapeDtypeStruct((B, D), jnp.float32),
        ),
        mesh=mesh,
        scratch_types=[
            pltpu.VMEM((B_PER_W,), jnp.int32),
            pltpu.VMEM((B_PER_W,), jnp.int32),
            pltpu.VMEM((B_PER_W, D), jnp.float32),
            pltpu.VMEM((B_PER_W, D), jnp.float32),
            pltpu.SemaphoreType.DMA,
        ],
        compiler_params=pltpu.CompilerParams(use_tc_tiling_on_sc=False),
    )
    def gather(ut_hbm, pt_hbm, ui_hbm, pi_hbm, uo_hbm, po_hbm,
               uidx_v, pidx_v, urows_v, prows_v, sem):
        wid = lax.axis_index("s") * NC + lax.axis_index("c")
        base = wid * B_PER_W
        pltpu.sync_copy(ui_hbm.at[pl.ds(base, B_PER_W)], uidx_v)
        pltpu.sync_copy(pi_hbm.at[pl.ds(base, B_PER_W)], pidx_v)
        copies = []
        for j in range(N_CHUNKS):
            sl = pl.ds(j * IDX_CHUNK, IDX_CHUNK)
            copies.append(pltpu.async_copy(
                ut_hbm.at[uidx_v.at[sl]], urows_v.at[sl], sem))
            copies.append(pltpu.async_copy(
                pt_hbm.at[pidx_v.at[sl]], prows_v.at[sl], sem))
        for c in copies:
            c.wait()
        pltpu.sync_copy(urows_v, uo_hbm.at[pl.ds(base, B_PER_W)])
        pltpu.sync_copy(prows_v, po_hbm.at[pl.ds(base, B_PER_W)])

    return gather


_sc_gather = _make_sc_gather()

BLK = 1024


def _mlp_body(u_ref, p_ref, w1u_ref, w1p_ref, b1_ref, w2_ref, b2_ref,
              w3_ref, b3_ref, o_ref):
    h1 = (jnp.dot(u_ref[...], w1u_ref[...], preferred_element_type=jnp.float32)
          + jnp.dot(p_ref[...], w1p_ref[...], preferred_element_type=jnp.float32)
          + b1_ref[...])
    h1 = jnp.maximum(h1, 0.0)
    h2 = jnp.dot(h1, w2_ref[...], preferred_element_type=jnp.float32) + b2_ref[...]
    h2 = jnp.maximum(h2, 0.0)
    o_ref[...] = jnp.sum(h2 * w3_ref[...], axis=1) + b3_ref[0, 0]


def _mlp(u, p, w1u, w1p, b1, w2, b2, w3, b3):
    grid = (B // BLK,)
    full = lambda i: (0, 0)
    return pl.pallas_call(
        _mlp_body,
        out_shape=jax.ShapeDtypeStruct((B,), jnp.float32),
        grid=grid,
        in_specs=[
            pl.BlockSpec((BLK, D), lambda i: (i, 0)),
            pl.BlockSpec((BLK, D), lambda i: (i, 0)),
            pl.BlockSpec((D, 64), full),
            pl.BlockSpec((D, 64), full),
            pl.BlockSpec((1, 64), full),
            pl.BlockSpec((64, 32), full),
            pl.BlockSpec((1, 32), full),
            pl.BlockSpec((1, 32), full),
            pl.BlockSpec((1, 1), full),
        ],
        out_specs=pl.BlockSpec((BLK,), lambda i: (i,)),
    )(u, p, w1u, w1p, b1, w2, b2, w3, b3)


def kernel(user_tensor, product_tensor, user_table, product_table,
           W1, b1, W2, b2, W3, b3):
    u_rows, p_rows = _sc_gather(user_table, product_table,
                                user_tensor.astype(jnp.int32),
                                product_tensor.astype(jnp.int32))
    w1u = W1[:, :D].T
    w1p = W1[:, D:].T
    return _mlp(u_rows, p_rows, w1u, w1p, b1.reshape(1, 64),
                W2.T, b2.reshape(1, 32), W3.reshape(1, 32),
                b3.reshape(1, 1))
